# baseline (speedup 1.0000x reference)
import jax
import jax.numpy as jnp
from jax.experimental import pallas as pl

NUM_NODES = 1000
FEAT = 64
HID = 64
OUT = 32
BATCH = 16
NTOT = BATCH * NUM_NODES


def _bf16_dot(a, w):
    return jnp.dot(a.astype(jnp.bfloat16), w.astype(jnp.bfloat16),
                   preferred_element_type=jnp.float32)


def _gcn_kernel(x_ref, wb_ref, out_ref):
    wb = wb_ref[...]
    h = x_ref[...]
    m0 = jnp.mean(x_ref[0:NUM_NODES], axis=0, keepdims=True)
    row = jax.lax.broadcasted_iota(jnp.int32, (NTOT, 1), 0)
    h = jnp.where(row < NUM_NODES, m0, h)
    h = jnp.maximum(_bf16_dot(h, wb[0:FEAT, 0:HID]) + wb[FEAT:FEAT + 1, 0:HID], 0.0)
    h = jnp.maximum(_bf16_dot(h, wb[0:HID, HID:2 * HID])
                    + wb[FEAT:FEAT + 1, HID:2 * HID], 0.0)
    bidx = jax.lax.broadcasted_iota(jnp.int32, (BATCH, NTOT), 0)
    nidx = jax.lax.broadcasted_iota(jnp.int32, (BATCH, NTOT), 1)
    pool = jnp.where(nidx // NUM_NODES == bidx, 1.0 / NUM_NODES, 0.0)
    pooled = jnp.dot(pool, h, preferred_element_type=jnp.float32)
    o = (jnp.dot(pooled, wb[0:HID, 2 * HID:3 * HID],
                 preferred_element_type=jnp.float32)
         + wb[FEAT:FEAT + 1, 2 * HID:3 * HID])
    out_ref[...] = (
        jnp.dot(o, wb[0:HID, 3 * HID:3 * HID + OUT],
                preferred_element_type=jnp.float32)
        + wb[FEAT:FEAT + 1, 3 * HID:3 * HID + OUT]
    )


@jax.jit
def _run(x, W1, b1, W2, b2, W3, b3, Wl, bl):
    B = x.shape[0]
    x = x.astype(jnp.float32).reshape(B * NUM_NODES, FEAT)
    wrow = jnp.concatenate(
        [W1, W2, W3, jnp.pad(Wl, ((0, 0), (0, HID - OUT)))], axis=1)
    brow = jnp.concatenate(
        [b1, b2, b3, jnp.pad(bl, (0, HID - OUT))]).reshape(1, 4 * HID)
    wb = jnp.concatenate([wrow, brow], axis=0)
    return pl.pallas_call(
        _gcn_kernel,
        out_shape=jax.ShapeDtypeStruct((B, OUT), jnp.float32),
    )(x, wb)


def kernel(x, W1, b1, W2, b2, W3, b3, Wl, bl, batch_size=BATCH, device=0):
    return _run(x, W1, b1, W2, b2, W3, b3, Wl, bl)


# speedup vs baseline: 1.1195x; 1.1195x over previous
import jax
import jax.numpy as jnp
from jax.experimental import pallas as pl

NUM_NODES = 1000
FEAT = 64
HID = 64
OUT = 32
BATCH = 16
NTOT = BATCH * NUM_NODES


def _gcn_kernel(x_ref, w1_ref, b1_ref, w2_ref, b2_ref, w3_ref, b3_ref,
                wl_ref, bl_ref, out_ref):
    h = x_ref[...]
    m0 = jnp.mean(x_ref[0:NUM_NODES], axis=0, keepdims=True)
    row = jax.lax.broadcasted_iota(jnp.int32, (NTOT, 1), 0)
    h = jnp.where(row < NUM_NODES, m0, h)
    h = jnp.dot(h, w1_ref[...], preferred_element_type=jnp.float32) + b1_ref[...]
    h = jnp.maximum(h, 0.0)
    h = jnp.dot(h, w2_ref[...], preferred_element_type=jnp.float32) + b2_ref[...]
    h = jnp.maximum(h, 0.0)
    bidx = jax.lax.broadcasted_iota(jnp.int32, (BATCH, NTOT), 0)
    nidx = jax.lax.broadcasted_iota(jnp.int32, (BATCH, NTOT), 1)
    pool = jnp.where(nidx // NUM_NODES == bidx, 1.0 / NUM_NODES, 0.0)
    pooled = jnp.dot(pool, h, preferred_element_type=jnp.float32)
    o = jnp.dot(pooled, w3_ref[...], preferred_element_type=jnp.float32) + b3_ref[...]
    out_ref[...] = (
        jnp.dot(o, wl_ref[...], preferred_element_type=jnp.float32) + bl_ref[...]
    )


@jax.jit
def _run(x, W1, b1, W2, b2, W3, b3, Wl, bl):
    B = x.shape[0]
    x = x.astype(jnp.float32).reshape(B * NUM_NODES, FEAT)
    b1 = b1.reshape(1, HID)
    b2 = b2.reshape(1, HID)
    b3 = b3.reshape(1, HID)
    bl = bl.reshape(1, OUT)
    return pl.pallas_call(
        _gcn_kernel,
        out_shape=jax.ShapeDtypeStruct((B, OUT), jnp.float32),
    )(x, W1, b1, W2, b2, W3, b3, Wl, bl)


def kernel(x, W1, b1, W2, b2, W3, b3, Wl, bl, batch_size=BATCH, device=0):
    return _run(x, W1, b1, W2, b2, W3, b3, Wl, bl)
